# Initial kernel scaffold; baseline (speedup 1.0000x reference)
import jax
import jax.numpy as jnp
from jax.experimental import pallas as pl


def kernel(x, edge_index, new_edge_indexs, W_mlp, b_mlp, W1, b1, W2, b2, Wl, bl, att, Wc, bc):
    raise NotImplementedError("write your pallas kernel here")


# speedup vs baseline: 8.2033x; 8.2033x over previous
import functools

import jax
import jax.numpy as jnp
from jax import lax
from jax.experimental import pallas as pl
from jax.experimental.pallas import tpu as pltpu
from jax.experimental.pallas import tpu_sc as plsc

N = 10000
E = 320000
D = 128
OUT = 64
HOPS = 3

NC = 2
NS = 16
NW = NC * NS
K = 128
E_PAD = NW * 80 * K
CPW = E_PAD // (NW * K)
GROUPS = CPW // 8
ROWS_PER_SET = E_PAD // K
N_PAD = 10240
RPT = N_PAD // NS
BLK = 1024
GRID = N_PAD // BLK

_MESH = dict(core_axis_name="c", subcore_axis_name="s", num_cores=NC,
             num_subcores=NS)


def _f32(*shape):
    return jax.ShapeDtypeStruct(shape, jnp.float32)


@functools.cache
def _get_sc_hist():
    return functools.partial(
        pl.kernel,
        out_type=_f32(NC * 4 * N_PAD, D),
        mesh=plsc.VectorSubcoreMesh(**_MESH),
        scratch_types=[
            pltpu.VMEM_SHARED((N_PAD, D), jnp.float32),
            pltpu.VMEM((8, K), jnp.int32),
            pltpu.VMEM((K, D), jnp.float32),
            pltpu.VMEM((K, D), jnp.float32),
        ],
    )(_sc_hist_body)


def _sc_hist_body(dsts_hbm, ones_hbm, zr_hbm, cnt_out, acc, dbuf, onesb, zb):
    c = lax.axis_index("c")
    s = lax.axis_index("s")
    wid = s * NC + c
    pltpu.sync_copy(ones_hbm, onesb)
    pltpu.sync_copy(zr_hbm, zb)
    for k in range(4):
        for t in range(RPT // K):
            pltpu.sync_copy(zb, acc.at[pl.ds(s * RPT + t * K, K)])
        plsc.subcore_barrier()
        base = k * ROWS_PER_SET + wid * CPW

        def group(g, _):
            pltpu.sync_copy(dsts_hbm.at[pl.ds(base + g * 8, 8)], dbuf)
            for j in range(8):
                pltpu.sync_copy(onesb, acc.at[dbuf.at[j]], add=True)
            return 0

        lax.fori_loop(0, GROUPS, group, 0)
        plsc.subcore_barrier()

        def copyout(t, _):
            pltpu.sync_copy(acc.at[pl.ds(s * RPT + t * K, K)], zb)
            pltpu.sync_copy(zb, cnt_out.at[
                pl.ds((c * 4 + k) * N_PAD + s * RPT + t * K, K)])
            return 0

        lax.fori_loop(0, RPT // K, copyout, 0)
        pltpu.sync_copy(zr_hbm, zb)


@functools.cache
def _make_scatter(nsets):
    def scatter(srcs_hbm, dsts_hbm, zr_hbm, *rest):
        tables = rest[:nsets]
        outs = rest[nsets:2 * nsets]
        acc, sbuf, dbuf, rows, zb, sem = rest[2 * nsets:]
        c = lax.axis_index("c")
        s = lax.axis_index("s")
        wid = s * NC + c
        pltpu.sync_copy(zr_hbm, zb)
        for k in range(nsets):
            for j in range(RPT // K):
                pltpu.sync_copy(zb, acc.at[pl.ds(s * RPT + j * K, K)])
            plsc.subcore_barrier()
            base = k * ROWS_PER_SET + wid * CPW

            def group(g, _):
                pltpu.sync_copy(srcs_hbm.at[pl.ds(base + g * 8, 8)], sbuf)
                pltpu.sync_copy(dsts_hbm.at[pl.ds(base + g * 8, 8)], dbuf)
                for j in range(8):
                    pltpu.async_copy(tables[k].at[sbuf.at[j]], rows,
                                     sem).wait()
                    pltpu.sync_copy(rows, acc.at[dbuf.at[j]], add=True)
                return 0

            lax.fori_loop(0, GROUPS, group, 0)
            plsc.subcore_barrier()

            def copyout(t, _):
                pltpu.sync_copy(acc.at[pl.ds(s * RPT + t * K, K)], rows)
                pltpu.sync_copy(
                    rows, outs[k].at[pl.ds(c * N_PAD + s * RPT + t * K, K)])
                return 0

            lax.fori_loop(0, RPT // K, copyout, 0)

    return functools.partial(
        pl.kernel,
        out_type=tuple(_f32(NC * N_PAD, D) for _ in range(nsets)),
        mesh=plsc.VectorSubcoreMesh(**_MESH),
        scratch_types=[
            pltpu.VMEM_SHARED((N_PAD, D), jnp.float32),
            pltpu.VMEM((8, K), jnp.int32),
            pltpu.VMEM((8, K), jnp.int32),
            pltpu.VMEM((K, D), jnp.float32),
            pltpu.VMEM((K, D), jnp.float32),
            pltpu.SemaphoreType.DMA,
        ],
    )(scatter)


def _tc1_body(x_ref, w_ref, bm_ref, cnt_ref,
              hp0_ref, hp1_ref, hp2_ref, hp3_ref, mlp_ref, dinv_ref):
    h = jnp.dot(x_ref[...], w_ref[...], preferred_element_type=jnp.float32)
    hp_refs = (hp0_ref, hp1_ref, hp2_ref, hp3_ref)
    dinv_cols = []
    for k in range(4):
        deg = cnt_ref[0, k, :, 0] + cnt_ref[1, k, :, 0] + 1.0
        dinv = lax.rsqrt(deg)
        dinv_cols.append(dinv)
        hp_refs[k][...] = h[:, D * k:D * (k + 1)] * dinv[:, None]
    dinv_ref[...] = jnp.stack(dinv_cols, axis=1)
    mlp_ref[...] = jnp.maximum(h[:, 4 * D:] + bm_ref[...], 0.0)


def _tc1(x, wcat, bm, cnt):
    return pl.pallas_call(
        _tc1_body,
        grid=(GRID,),
        in_specs=[
            pl.BlockSpec((BLK, D), lambda i: (i, 0)),
            pl.BlockSpec((D, 5 * D), lambda i: (0, 0)),
            pl.BlockSpec((1, D), lambda i: (0, 0)),
            pl.BlockSpec((NC, 4, BLK, D), lambda i: (0, 0, i, 0)),
        ],
        out_specs=[pl.BlockSpec((BLK, D), lambda i: (i, 0))] * 5
        + [pl.BlockSpec((BLK, 4), lambda i: (i, 0))],
        out_shape=[_f32(N_PAD, D)] * 5 + [_f32(N_PAD, 4)],
    )(x, wcat, bm, cnt)


def _tc2_body(acc_ref, hp0_ref, dinv_ref, b1_ref, w2_ref, h2p_ref):
    a = acc_ref[...]
    d0 = dinv_ref[...][:, 0][:, None]
    ec1 = jnp.maximum(d0 * (a[0] + a[1] + hp0_ref[...]) + b1_ref[...], 0.0)
    h2p_ref[...] = jnp.dot(ec1, w2_ref[...],
                           preferred_element_type=jnp.float32) * d0


def _tc2(acc1, hp0, dinv, b1, w2):
    return pl.pallas_call(
        _tc2_body,
        grid=(GRID,),
        in_specs=[
            pl.BlockSpec((2, BLK, D), lambda i: (0, i, 0)),
            pl.BlockSpec((BLK, D), lambda i: (i, 0)),
            pl.BlockSpec((BLK, 4), lambda i: (i, 0)),
            pl.BlockSpec((1, D), lambda i: (0, 0)),
            pl.BlockSpec((D, D), lambda i: (0, 0)),
        ],
        out_specs=pl.BlockSpec((BLK, D), lambda i: (i, 0)),
        out_shape=_f32(N_PAD, D),
    )(acc1, hp0, dinv, b1, w2)


def _tc3_body(acc2_ref, h2p_ref, b2_ref, a1_ref, hp1_ref, a2_ref, hp2_ref,
              a3_ref, hp3_ref, bl_ref, mlp_ref, dinv_ref, wc_ref, bc_ref,
              out_ref):
    d = dinv_ref[...]
    wc = wc_ref[...]
    bl = bl_ref[...]
    a = acc2_ref[...]
    ec = jnp.maximum(d[:, 0][:, None] * (a[0] + a[1] + h2p_ref[...])
                     + b2_ref[...], 0.0)
    z = jnp.dot(ec, wc[3], preferred_element_type=jnp.float32)
    for j, (ar, hr) in enumerate(((a1_ref, hp1_ref), (a2_ref, hp2_ref),
                                  (a3_ref, hp3_ref))):
        av = ar[...]
        o = jnp.maximum(d[:, j + 1][:, None] * (av[0] + av[1] + hr[...])
                        + bl[j], 0.0)
        z = z + jnp.dot(o, wc[j], preferred_element_type=jnp.float32)
    z = z + jnp.dot(mlp_ref[...], wc[4], preferred_element_type=jnp.float32)
    z = z + bc_ref[...]
    zm = jnp.max(z, axis=-1, keepdims=True)
    ze = z - zm
    out_ref[...] = ze - jnp.log(jnp.sum(jnp.exp(ze), axis=-1, keepdims=True))


def _tc3(acc2, h2p, b2, a1, hp1, a2, hp2, a3, hp3, bl, mlpo, dinv, wcp, bc):
    part = pl.BlockSpec((2, BLK, D), lambda i: (0, i, 0))
    row = pl.BlockSpec((BLK, D), lambda i: (i, 0))
    return pl.pallas_call(
        _tc3_body,
        grid=(GRID,),
        in_specs=[
            part, row,
            pl.BlockSpec((1, D), lambda i: (0, 0)),
            part, row, part, row, part, row,
            pl.BlockSpec((HOPS, D), lambda i: (0, 0)),
            row,
            pl.BlockSpec((BLK, 4), lambda i: (i, 0)),
            pl.BlockSpec((5, D, OUT), lambda i: (0, 0, 0)),
            pl.BlockSpec((1, OUT), lambda i: (0, 0)),
        ],
        out_specs=pl.BlockSpec((BLK, OUT), lambda i: (i, 0)),
        out_shape=_f32(N_PAD, OUT),
    )(acc2, h2p, b2, a1, hp1, a2, hp2, a3, hp3, bl, mlpo, dinv, wcp, bc)


@jax.jit
def kernel(x, edge_index, new_edge_indexs, W_mlp, b_mlp, W1, b1, W2, b2,
           Wl, bl, att, Wc, bc):
    f32 = jnp.float32
    srcs = jnp.concatenate([edge_index[0][None], new_edge_indexs[:, 0]], 0)
    dsts = jnp.concatenate([edge_index[1][None], new_edge_indexs[:, 1]], 0)
    srcs = jnp.pad(srcs, ((0, 0), (0, E_PAD - E)))
    dsts = jnp.pad(dsts, ((0, 0), (0, E_PAD - E)), constant_values=N)
    srcs_r = srcs.reshape(4 * ROWS_PER_SET, K)
    dsts_r = dsts.reshape(4 * ROWS_PER_SET, K)

    zr = jnp.zeros((K, D), f32)

    wcat = jnp.concatenate([W1, Wl[0], Wl[1], Wl[2], W_mlp], axis=1)
    m = jax.nn.softmax(att)
    mscale = jnp.stack([m[1], m[2], m[3], m[1], m[0]])
    wcp = Wc.reshape(5, D, OUT) * mscale[:, None, None]

    ones_rows = jnp.ones((K, D), f32)
    xp = jnp.pad(x, ((0, N_PAD - N), (0, 0)))

    cnt = _get_sc_hist()(dsts_r, ones_rows, zr).reshape(NC, 4, N_PAD, D)

    hp0, hp1, hp2, hp3, mlpo, dinv = _tc1(
        xp, wcat, b_mlp.reshape(1, D), cnt)

    o0, o1, o2, o3 = _make_scatter(4)(srcs_r, dsts_r, zr, hp0, hp1, hp2, hp3)

    h2p = _tc2(o0.reshape(NC, N_PAD, D), hp0, dinv, b1.reshape(1, D), W2)

    (o4,) = _make_scatter(1)(srcs_r, dsts_r, zr, h2p)

    z = _tc3(o4.reshape(NC, N_PAD, D), h2p, b2.reshape(1, D),
             o1.reshape(NC, N_PAD, D), hp1,
             o2.reshape(NC, N_PAD, D), hp2,
             o3.reshape(NC, N_PAD, D), hp3,
             bl, mlpo, dinv, wcp, bc.reshape(1, OUT))
    return z[:N]


# speedup vs baseline: 8.7195x; 1.0629x over previous
import functools

import jax
import jax.numpy as jnp
from jax import lax
from jax.experimental import pallas as pl
from jax.experimental.pallas import tpu as pltpu
from jax.experimental.pallas import tpu_sc as plsc

N = 10000
E = 320000
D = 128
OUT = 64
HOPS = 3

NC = 2
NS = 16
NW = NC * NS
K = 128
E_PAD = NW * 80 * K
CPW = E_PAD // (NW * K)
GROUPS = CPW // 8
ROWS_PER_SET = E_PAD // K
N_PAD = 10240
RPT = N_PAD // NS
BLK = 1024
GRID = N_PAD // BLK

_MESH = dict(core_axis_name="c", subcore_axis_name="s", num_cores=NC,
             num_subcores=NS)


def _f32(*shape):
    return jax.ShapeDtypeStruct(shape, jnp.float32)


@functools.cache
def _get_sc_hist():
    return functools.partial(
        pl.kernel,
        out_type=_f32(NC * 4 * N_PAD, D),
        mesh=plsc.VectorSubcoreMesh(**_MESH),
        scratch_types=[
            pltpu.VMEM_SHARED((N_PAD, D), jnp.float32),
            pltpu.VMEM((8, K), jnp.int32),
            pltpu.VMEM((K, D), jnp.float32),
            pltpu.VMEM((K, D), jnp.float32),
            pltpu.SemaphoreType.DMA,
        ],
    )(_sc_hist_body)


def _sc_hist_body(dsts_hbm, ones_hbm, zr_hbm, cnt_out, acc, dbuf, onesb, zb,
                  sem):
    c = lax.axis_index("c")
    s = lax.axis_index("s")
    wid = s * NC + c
    pltpu.sync_copy(ones_hbm, onesb)
    pltpu.sync_copy(zr_hbm, zb)
    for k in range(4):
        for t in range(RPT // K):
            pltpu.sync_copy(zb, acc.at[pl.ds(s * RPT + t * K, K)])
        plsc.subcore_barrier()
        base = k * ROWS_PER_SET + wid * CPW

        def group(g, _):
            pltpu.sync_copy(dsts_hbm.at[pl.ds(base + g * 8, 8)], dbuf)
            descs = [pltpu.async_copy(onesb, acc.at[dbuf.at[j]], sem,
                                      add=True) for j in range(8)]
            for d in descs:
                d.wait()
            return 0

        lax.fori_loop(0, GROUPS, group, 0)
        plsc.subcore_barrier()

        def copyout(t, _):
            pltpu.sync_copy(acc.at[pl.ds(s * RPT + t * K, K)], zb)
            pltpu.sync_copy(zb, cnt_out.at[
                pl.ds((c * 4 + k) * N_PAD + s * RPT + t * K, K)])
            return 0

        lax.fori_loop(0, RPT // K, copyout, 0)
        pltpu.sync_copy(zr_hbm, zb)


@functools.cache
def _make_scatter(nsets):
    def scatter(srcs_hbm, dsts_hbm, zr_hbm, *rest):
        tables = rest[:nsets]
        outs = rest[nsets:2 * nsets]
        acc, sbuf, dbuf, rows0, rows1, sem = rest[2 * nsets:]
        rbufs = (rows0, rows1)
        c = lax.axis_index("c")
        s = lax.axis_index("s")
        wid = s * NC + c
        for k in range(nsets):
            pltpu.sync_copy(zr_hbm, rows0)
            for j in range(RPT // K):
                pltpu.sync_copy(rows0, acc.at[pl.ds(s * RPT + j * K, K)])
            plsc.subcore_barrier()
            base = k * ROWS_PER_SET + wid * CPW

            def group(g, _):
                pltpu.sync_copy(srcs_hbm.at[pl.ds(base + g * 8, 8)], sbuf)
                pltpu.sync_copy(dsts_hbm.at[pl.ds(base + g * 8, 8)], dbuf)
                d = pltpu.async_copy(tables[k].at[sbuf.at[0]], rbufs[0], sem)
                for j in range(8):
                    d.wait()
                    if j < 7:
                        d = pltpu.async_copy(tables[k].at[sbuf.at[j + 1]],
                                             rbufs[(j + 1) % 2], sem)
                    pltpu.sync_copy(rbufs[j % 2], acc.at[dbuf.at[j]],
                                    add=True)
                return 0

            lax.fori_loop(0, GROUPS, group, 0)
            plsc.subcore_barrier()

            def copyout(t, _):
                pltpu.sync_copy(acc.at[pl.ds(s * RPT + t * K, K)], rows0)
                pltpu.sync_copy(
                    rows0, outs[k].at[pl.ds(c * N_PAD + s * RPT + t * K, K)])
                return 0

            lax.fori_loop(0, RPT // K, copyout, 0)

    return functools.partial(
        pl.kernel,
        out_type=tuple(_f32(NC * N_PAD, D) for _ in range(nsets)),
        mesh=plsc.VectorSubcoreMesh(**_MESH),
        scratch_types=[
            pltpu.VMEM_SHARED((N_PAD, D), jnp.float32),
            pltpu.VMEM((8, K), jnp.int32),
            pltpu.VMEM((8, K), jnp.int32),
            pltpu.VMEM((K, D), jnp.float32),
            pltpu.VMEM((K, D), jnp.float32),
            pltpu.SemaphoreType.DMA,
        ],
    )(scatter)


def _tc1_body(x_ref, w_ref, bm_ref, cnt_ref,
              hp0_ref, hp1_ref, hp2_ref, hp3_ref, mlp_ref, dinv_ref):
    h = jnp.dot(x_ref[...], w_ref[...], preferred_element_type=jnp.float32)
    hp_refs = (hp0_ref, hp1_ref, hp2_ref, hp3_ref)
    dinv_cols = []
    for k in range(4):
        deg = cnt_ref[0, k, :, 0] + cnt_ref[1, k, :, 0] + 1.0
        dinv = lax.rsqrt(deg)
        dinv_cols.append(dinv)
        hp_refs[k][...] = h[:, D * k:D * (k + 1)] * dinv[:, None]
    dinv_ref[...] = jnp.stack(dinv_cols, axis=1)
    mlp_ref[...] = jnp.maximum(h[:, 4 * D:] + bm_ref[...], 0.0)


def _tc1(x, wcat, bm, cnt):
    return pl.pallas_call(
        _tc1_body,
        grid=(GRID,),
        in_specs=[
            pl.BlockSpec((BLK, D), lambda i: (i, 0)),
            pl.BlockSpec((D, 5 * D), lambda i: (0, 0)),
            pl.BlockSpec((1, D), lambda i: (0, 0)),
            pl.BlockSpec((NC, 4, BLK, D), lambda i: (0, 0, i, 0)),
        ],
        out_specs=[pl.BlockSpec((BLK, D), lambda i: (i, 0))] * 5
        + [pl.BlockSpec((BLK, 4), lambda i: (i, 0))],
        out_shape=[_f32(N_PAD, D)] * 5 + [_f32(N_PAD, 4)],
    )(x, wcat, bm, cnt)


def _tc2_body(acc_ref, hp0_ref, dinv_ref, b1_ref, w2_ref, h2p_ref):
    a = acc_ref[...]
    d0 = dinv_ref[...][:, 0][:, None]
    ec1 = jnp.maximum(d0 * (a[0] + a[1] + hp0_ref[...]) + b1_ref[...], 0.0)
    h2p_ref[...] = jnp.dot(ec1, w2_ref[...],
                           preferred_element_type=jnp.float32) * d0


def _tc2(acc1, hp0, dinv, b1, w2):
    return pl.pallas_call(
        _tc2_body,
        grid=(GRID,),
        in_specs=[
            pl.BlockSpec((2, BLK, D), lambda i: (0, i, 0)),
            pl.BlockSpec((BLK, D), lambda i: (i, 0)),
            pl.BlockSpec((BLK, 4), lambda i: (i, 0)),
            pl.BlockSpec((1, D), lambda i: (0, 0)),
            pl.BlockSpec((D, D), lambda i: (0, 0)),
        ],
        out_specs=pl.BlockSpec((BLK, D), lambda i: (i, 0)),
        out_shape=_f32(N_PAD, D),
    )(acc1, hp0, dinv, b1, w2)


def _tc3_body(acc2_ref, h2p_ref, b2_ref, a1_ref, hp1_ref, a2_ref, hp2_ref,
              a3_ref, hp3_ref, bl_ref, mlp_ref, dinv_ref, wc_ref, bc_ref,
              out_ref):
    d = dinv_ref[...]
    wc = wc_ref[...]
    bl = bl_ref[...]
    a = acc2_ref[...]
    ec = jnp.maximum(d[:, 0][:, None] * (a[0] + a[1] + h2p_ref[...])
                     + b2_ref[...], 0.0)
    z = jnp.dot(ec, wc[3], preferred_element_type=jnp.float32)
    for j, (ar, hr) in enumerate(((a1_ref, hp1_ref), (a2_ref, hp2_ref),
                                  (a3_ref, hp3_ref))):
        av = ar[...]
        o = jnp.maximum(d[:, j + 1][:, None] * (av[0] + av[1] + hr[...])
                        + bl[j], 0.0)
        z = z + jnp.dot(o, wc[j], preferred_element_type=jnp.float32)
    z = z + jnp.dot(mlp_ref[...], wc[4], preferred_element_type=jnp.float32)
    z = z + bc_ref[...]
    zm = jnp.max(z, axis=-1, keepdims=True)
    ze = z - zm
    out_ref[...] = ze - jnp.log(jnp.sum(jnp.exp(ze), axis=-1, keepdims=True))


def _tc3(acc2, h2p, b2, a1, hp1, a2, hp2, a3, hp3, bl, mlpo, dinv, wcp, bc):
    part = pl.BlockSpec((2, BLK, D), lambda i: (0, i, 0))
    row = pl.BlockSpec((BLK, D), lambda i: (i, 0))
    return pl.pallas_call(
        _tc3_body,
        grid=(GRID,),
        in_specs=[
            part, row,
            pl.BlockSpec((1, D), lambda i: (0, 0)),
            part, row, part, row, part, row,
            pl.BlockSpec((HOPS, D), lambda i: (0, 0)),
            row,
            pl.BlockSpec((BLK, 4), lambda i: (i, 0)),
            pl.BlockSpec((5, D, OUT), lambda i: (0, 0, 0)),
            pl.BlockSpec((1, OUT), lambda i: (0, 0)),
        ],
        out_specs=pl.BlockSpec((BLK, OUT), lambda i: (i, 0)),
        out_shape=_f32(N_PAD, OUT),
    )(acc2, h2p, b2, a1, hp1, a2, hp2, a3, hp3, bl, mlpo, dinv, wcp, bc)


@jax.jit
def kernel(x, edge_index, new_edge_indexs, W_mlp, b_mlp, W1, b1, W2, b2,
           Wl, bl, att, Wc, bc):
    f32 = jnp.float32
    srcs = jnp.concatenate([edge_index[0][None], new_edge_indexs[:, 0]], 0)
    dsts = jnp.concatenate([edge_index[1][None], new_edge_indexs[:, 1]], 0)
    srcs = jnp.pad(srcs, ((0, 0), (0, E_PAD - E)))
    dsts = jnp.pad(dsts, ((0, 0), (0, E_PAD - E)), constant_values=N)
    srcs_r = srcs.reshape(4 * ROWS_PER_SET, K)
    dsts_r = dsts.reshape(4 * ROWS_PER_SET, K)

    zr = jnp.zeros((K, D), f32)

    wcat = jnp.concatenate([W1, Wl[0], Wl[1], Wl[2], W_mlp], axis=1)
    m = jax.nn.softmax(att)
    mscale = jnp.stack([m[1], m[2], m[3], m[1], m[0]])
    wcp = Wc.reshape(5, D, OUT) * mscale[:, None, None]

    ones_rows = jnp.ones((K, D), f32)
    xp = jnp.pad(x, ((0, N_PAD - N), (0, 0)))

    cnt = _get_sc_hist()(dsts_r, ones_rows, zr).reshape(NC, 4, N_PAD, D)

    hp0, hp1, hp2, hp3, mlpo, dinv = _tc1(
        xp, wcat, b_mlp.reshape(1, D), cnt)

    o0, o1, o2, o3 = _make_scatter(4)(srcs_r, dsts_r, zr, hp0, hp1, hp2, hp3)

    h2p = _tc2(o0.reshape(NC, N_PAD, D), hp0, dinv, b1.reshape(1, D), W2)

    (o4,) = _make_scatter(1)(srcs_r, dsts_r, zr, h2p)

    z = _tc3(o4.reshape(NC, N_PAD, D), h2p, b2.reshape(1, D),
             o1.reshape(NC, N_PAD, D), hp1,
             o2.reshape(NC, N_PAD, D), hp2,
             o3.reshape(NC, N_PAD, D), hp3,
             bl, mlpo, dinv, wcp, bc.reshape(1, OUT))
    return z[:N]
